# baseline (speedup 1.0000x reference)
import functools

import jax
import jax.numpy as jnp
from jax import lax
from jax.experimental import pallas as pl
from jax.experimental.pallas import tpu as pltpu
from jax.experimental.pallas import tpu_sc as plsc

N = 50000
E = 1600000
D_IN = 128
D_H = 32
N_TAPS = 4
N_LAYERS = 2

BLK = 2000
NBLK = N // BLK
YPAD = 50048
BLK2 = 3128
NBLK2 = YPAD // BLK2

NTILES = 16
CH = 100
NB = 8
NROW2D = E // CH
NCH_TILE = NROW2D // NTILES
NGRP = NCH_TILE // NB
RPT = YPAD // NTILES


def _readin_body(xa_ref, xb_ref, w_ref, b_ref, h_ref, sum_ref, sq_ref):
    i = pl.program_id(0)
    ha = jnp.dot(xa_ref[...], w_ref[0], preferred_element_type=jnp.float32) + b_ref[0]
    hb = jnp.dot(xb_ref[...], w_ref[1], preferred_element_type=jnp.float32) + b_ref[1]
    h_ref[0] = ha
    h_ref[1] = hb
    ps = jnp.stack([jnp.sum(ha, 0), jnp.sum(hb, 0)])[:, None, :]
    pq = jnp.stack([jnp.sum(ha * ha, 0), jnp.sum(hb * hb, 0)])[:, None, :]

    @pl.when(i == 0)
    def _():
        sum_ref[...] = ps
        sq_ref[...] = pq

    @pl.when(i > 0)
    def _():
        sum_ref[...] += ps
        sq_ref[...] += pq


def _readin(x_a, x_b, W_in, b_in):
    return pl.pallas_call(
        _readin_body,
        grid=(NBLK,),
        in_specs=[
            pl.BlockSpec((BLK, D_IN), lambda i: (i, 0)),
            pl.BlockSpec((BLK, D_IN), lambda i: (i, 0)),
            pl.BlockSpec((2, D_IN, D_H), lambda i: (0, 0, 0)),
            pl.BlockSpec((2, D_H), lambda i: (0, 0)),
        ],
        out_specs=[
            pl.BlockSpec((2, BLK, D_H), lambda i: (0, i, 0)),
            pl.BlockSpec((2, 1, D_H), lambda i: (0, 0, 0)),
            pl.BlockSpec((2, 1, D_H), lambda i: (0, 0, 0)),
        ],
        out_shape=[
            jax.ShapeDtypeStruct((2, YPAD, D_H), jnp.float32),
            jax.ShapeDtypeStruct((2, 1, D_H), jnp.float32),
            jax.ShapeDtypeStruct((2, 1, D_H), jnp.float32),
        ],
    )(x_a, x_b, W_in, b_in)


def _norm_body(h_ref, sum_ref, sq_ref, g_ref, bt_ref, y_ref):
    mu = sum_ref[0] / N
    var = sq_ref[0] / N - mu * mu
    inv = lax.rsqrt(var + 1e-5)
    v = (h_ref[0] - mu) * inv * g_ref[0] + bt_ref[0]
    y_ref[...] = jnp.where(v >= 0, v, 0.01 * v)


def _norm_leaky(h, ssum, ssq, g, bt):
    return pl.pallas_call(
        _norm_body,
        grid=(2, NBLK2),
        in_specs=[
            pl.BlockSpec((1, BLK2, D_H), lambda t, i: (t, i, 0)),
            pl.BlockSpec((1, 1, D_H), lambda t, i: (t, 0, 0)),
            pl.BlockSpec((1, 1, D_H), lambda t, i: (t, 0, 0)),
            pl.BlockSpec((1, 1, D_H), lambda t, i: (t, 0, 0)),
            pl.BlockSpec((1, 1, D_H), lambda t, i: (t, 0, 0)),
        ],
        out_specs=pl.BlockSpec((BLK2, D_H), lambda t, i: (t * NBLK2 + i, 0)),
        out_shape=jax.ShapeDtypeStruct((2 * YPAD, D_H), jnp.float32),
    )(h, ssum, ssq, g[:, None, :], bt[:, None, :])


def _comb_body(h_ref, y0, y1, y2, y3, y4, w_ref, b_ref, ho_ref, sum_ref, sq_ref):
    i = pl.program_id(1)
    acc = h_ref[0]
    for k, y in enumerate((y0, y1, y2, y3, y4)):
        acc = acc + jnp.dot(y[...], w_ref[k, 0], preferred_element_type=jnp.float32)
    acc = acc + jnp.sum(b_ref[...], axis=0)[0]
    ho_ref[0] = acc
    rows = i * BLK2 + lax.broadcasted_iota(jnp.int32, (BLK2, 1), 0)
    m = rows < N
    ps = jnp.sum(jnp.where(m, acc, 0.0), 0)[None, None, :]
    pq = jnp.sum(jnp.where(m, acc * acc, 0.0), 0)[None, None, :]

    @pl.when(i == 0)
    def _():
        sum_ref[...] = ps
        sq_ref[...] = pq

    @pl.when(i > 0)
    def _():
        sum_ref[...] += ps
        sq_ref[...] += pq


def _combine(h, ys, Wt, bt):
    yflat = pl.BlockSpec((BLK2, D_H), lambda t, i: (t * NBLK2 + i, 0))
    return pl.pallas_call(
        _comb_body,
        grid=(2, NBLK2),
        in_specs=[pl.BlockSpec((1, BLK2, D_H), lambda t, i: (t, i, 0))]
        + [yflat] * 5
        + [
            pl.BlockSpec((N_TAPS + 1, 1, D_H, D_H), lambda t, i: (0, t, 0, 0)),
            pl.BlockSpec((N_TAPS + 1, 1, 1, D_H), lambda t, i: (0, t, 0, 0)),
        ],
        out_specs=[
            pl.BlockSpec((1, BLK2, D_H), lambda t, i: (t, i, 0)),
            pl.BlockSpec((1, 1, D_H), lambda t, i: (t, 0, 0)),
            pl.BlockSpec((1, 1, D_H), lambda t, i: (t, 0, 0)),
        ],
        out_shape=[
            jax.ShapeDtypeStruct((2, YPAD, D_H), jnp.float32),
            jax.ShapeDtypeStruct((2, 1, D_H), jnp.float32),
            jax.ShapeDtypeStruct((2, 1, D_H), jnp.float32),
        ],
    )(h, *ys, Wt, bt[:, :, None, :])


def _readout_body(h_ref, w_ref, b_ref, o_ref):
    o_ref[0] = jnp.dot(h_ref[0], w_ref[0],
                       preferred_element_type=jnp.float32) + b_ref[0, 0]


def _readout(h, W_out, b_out):
    return pl.pallas_call(
        _readout_body,
        grid=(2, NBLK2),
        in_specs=[
            pl.BlockSpec((1, BLK2, D_H), lambda t, i: (t, i, 0)),
            pl.BlockSpec((1, D_H, D_IN), lambda t, i: (t, 0, 0)),
            pl.BlockSpec((1, 1, D_IN), lambda t, i: (t, 0, 0)),
        ],
        out_specs=pl.BlockSpec((1, BLK2, D_IN), lambda t, i: (t, i, 0)),
        out_shape=jax.ShapeDtypeStruct((2, N, D_IN), jnp.float32),
    )(h, W_out, b_out[:, None, :])


_sc_mesh = plsc.VectorSubcoreMesh(core_axis_name="c", subcore_axis_name="s")


@functools.partial(
    pl.kernel,
    out_type=jax.ShapeDtypeStruct((2 * YPAD, D_H), jnp.float32),
    mesh=_sc_mesh,
    scratch_types=[
        pltpu.VMEM((2, NB, 2, CH), jnp.int32),
        pltpu.VMEM((NB, CH, D_H), jnp.float32),
        pltpu.VMEM_SHARED((YPAD, D_H), jnp.float32),
    ] + [pltpu.SemaphoreType.DMA] * (2 * NB + 1),
    compiler_params=pltpu.CompilerParams(use_tc_tiling_on_sc=False),
)
def _seg_pair(y2_hbm, sdx_hbm, zeros_hbm, out_hbm, idx, rows, acc, *sems):
    gsem = sems[:NB]
    ssem = sems[NB:2 * NB]
    isem = sems[2 * NB]
    c = lax.axis_index("c")
    s = lax.axis_index("s")
    row0 = s * RPT
    crow0 = c * NROW2D + s * NCH_TILE

    def idx_copy(g):
        p = g & 1
        sl = pl.ds(crow0 + g * NB, NB)
        return pltpu.make_async_copy(sdx_hbm.at[sl, :, :], idx.at[p], isem)

    def idx_start(g):
        idx_copy(g).start()

    def idx_wait(g):
        idx_copy(g).wait()

    def fire_gathers(g):
        p = g & 1
        for b in range(NB):
            pltpu.async_copy(y2_hbm.at[idx.at[p, b, 0]], rows.at[b], gsem[b])

    def wait_and_scatter(g):
        p = g & 1
        for b in range(NB):
            pltpu.make_async_copy(
                y2_hbm.at[idx.at[p, b, 0]], rows.at[b], gsem[b]).wait()
            pltpu.async_copy(rows.at[b], acc.at[idx.at[p, b, 1]], ssem[b],
                             add=True)

    def drain_scatters(g):
        p = g & 1
        for b in range(NB):
            pltpu.make_async_copy(
                rows.at[b], acc.at[idx.at[p, b, 1]], ssem[b]).wait()

    idx_start(0)
    pltpu.sync_copy(zeros_hbm.at[pl.ds(row0, RPT), :],
                    acc.at[pl.ds(row0, RPT), :])
    plsc.subcore_barrier()

    idx_wait(0)
    idx_start(1)
    fire_gathers(0)
    wait_and_scatter(0)

    @pl.loop(1, NGRP - 1)
    def _(g):
        drain_scatters(g - 1)
        idx_wait(g)
        idx_start(g + 1)
        fire_gathers(g)
        wait_and_scatter(g)

    drain_scatters(NGRP - 2)
    idx_wait(NGRP - 1)
    fire_gathers(NGRP - 1)
    wait_and_scatter(NGRP - 1)
    drain_scatters(NGRP - 1)

    plsc.subcore_barrier()

    pltpu.sync_copy(acc.at[pl.ds(row0, RPT), :],
                    out_hbm.at[pl.ds(c * YPAD + row0, RPT), :])


def _seg_tap(y2, sdx, zeros):
    return _seg_pair(y2, sdx, zeros)


def kernel(x_a, x_b, edge_index_ab, edge_index_ba, W_in, b_in, W_taps,
           b_taps, gamma, beta, W_out, b_out):
    eab = lax.optimization_barrier(edge_index_ab.reshape(2 * E))
    eba = lax.optimization_barrier(edge_index_ba.reshape(2 * E))
    src1 = jnp.concatenate([eba[:E] + YPAD,
                            eab[:E]]).reshape(2 * NROW2D, CH)
    dst1 = jnp.concatenate([eba[E:], eab[E:]]).reshape(2 * NROW2D, CH)
    sdx = jnp.stack([src1, dst1], axis=1)
    zeros = jnp.zeros((YPAD, D_H), jnp.float32)

    h, ssum, ssq = _readin(x_a, x_b, W_in, b_in)
    for l in range(N_LAYERS):
        y = _norm_leaky(h, ssum, ssq, gamma[l], beta[l])
        ys = [y]
        for _ in range(N_TAPS):
            ys.append(_seg_tap(ys[-1], sdx, zeros))
        h, ssum, ssq = _combine(h, ys, W_taps[l], b_taps[l])
    return _readout(h, W_out, b_out)


# speedup vs baseline: 1.0349x; 1.0349x over previous
import functools

import jax
import jax.numpy as jnp
from jax import lax
from jax.experimental import pallas as pl
from jax.experimental.pallas import tpu as pltpu
from jax.experimental.pallas import tpu_sc as plsc

N = 50000
E = 1600000
D_IN = 128
D_H = 32
N_TAPS = 4
N_LAYERS = 2

BLK = 2000
NBLK = N // BLK
YPAD = 50048
BLK2 = 3128
NBLK2 = YPAD // BLK2

NTILES = 16
CH = 100
NB = 8
NROW2D = E // CH
NCH_TILE = NROW2D // NTILES
NGRP = NCH_TILE // NB
RPT = YPAD // NTILES


def _readin_body(xa_ref, xb_ref, w_ref, b_ref, h_ref, sum_ref, sq_ref):
    i = pl.program_id(0)
    ha = jnp.dot(xa_ref[...], w_ref[0], preferred_element_type=jnp.float32) + b_ref[0]
    hb = jnp.dot(xb_ref[...], w_ref[1], preferred_element_type=jnp.float32) + b_ref[1]
    h_ref[0] = ha
    h_ref[1] = hb
    ps = jnp.stack([jnp.sum(ha, 0), jnp.sum(hb, 0)])[:, None, :]
    pq = jnp.stack([jnp.sum(ha * ha, 0), jnp.sum(hb * hb, 0)])[:, None, :]

    @pl.when(i == 0)
    def _():
        sum_ref[...] = ps
        sq_ref[...] = pq

    @pl.when(i > 0)
    def _():
        sum_ref[...] += ps
        sq_ref[...] += pq


def _readin(x_a, x_b, W_in, b_in):
    return pl.pallas_call(
        _readin_body,
        grid=(NBLK,),
        in_specs=[
            pl.BlockSpec((BLK, D_IN), lambda i: (i, 0)),
            pl.BlockSpec((BLK, D_IN), lambda i: (i, 0)),
            pl.BlockSpec((2, D_IN, D_H), lambda i: (0, 0, 0)),
            pl.BlockSpec((2, D_H), lambda i: (0, 0)),
        ],
        out_specs=[
            pl.BlockSpec((2, BLK, D_H), lambda i: (0, i, 0)),
            pl.BlockSpec((2, 1, D_H), lambda i: (0, 0, 0)),
            pl.BlockSpec((2, 1, D_H), lambda i: (0, 0, 0)),
        ],
        out_shape=[
            jax.ShapeDtypeStruct((2, YPAD, D_H), jnp.float32),
            jax.ShapeDtypeStruct((2, 1, D_H), jnp.float32),
            jax.ShapeDtypeStruct((2, 1, D_H), jnp.float32),
        ],
    )(x_a, x_b, W_in, b_in)


def _norm_body(h_ref, sum_ref, sq_ref, g_ref, bt_ref, y_ref):
    mu = sum_ref[0] / N
    var = sq_ref[0] / N - mu * mu
    inv = lax.rsqrt(var + 1e-5)
    v = (h_ref[0] - mu) * inv * g_ref[0] + bt_ref[0]
    y_ref[...] = jnp.where(v >= 0, v, 0.01 * v)


def _norm_leaky(h, ssum, ssq, g, bt):
    return pl.pallas_call(
        _norm_body,
        grid=(2, NBLK2),
        in_specs=[
            pl.BlockSpec((1, BLK2, D_H), lambda t, i: (t, i, 0)),
            pl.BlockSpec((1, 1, D_H), lambda t, i: (t, 0, 0)),
            pl.BlockSpec((1, 1, D_H), lambda t, i: (t, 0, 0)),
            pl.BlockSpec((1, 1, D_H), lambda t, i: (t, 0, 0)),
            pl.BlockSpec((1, 1, D_H), lambda t, i: (t, 0, 0)),
        ],
        out_specs=pl.BlockSpec((BLK2, D_H), lambda t, i: (t * NBLK2 + i, 0)),
        out_shape=jax.ShapeDtypeStruct((2 * YPAD, D_H), jnp.float32),
    )(h, ssum, ssq, g[:, None, :], bt[:, None, :])


def _zinit_body(y_ref, w_ref, b_ref, zo_ref):
    zo_ref[0] = jnp.dot(y_ref[...], w_ref[0],
                        preferred_element_type=jnp.float32) \
        + jnp.sum(b_ref[...], axis=0)[0]


def _zinit(y0, W0, bt):
    return pl.pallas_call(
        _zinit_body,
        grid=(2, NBLK2),
        in_specs=[
            pl.BlockSpec((BLK2, D_H), lambda t, i: (t * NBLK2 + i, 0)),
            pl.BlockSpec((1, D_H, D_H), lambda t, i: (t, 0, 0)),
            pl.BlockSpec((N_TAPS + 1, 1, 1, D_H), lambda t, i: (0, t, 0, 0)),
        ],
        out_specs=pl.BlockSpec((1, BLK2, D_H), lambda t, i: (t, i, 0)),
        out_shape=jax.ShapeDtypeStruct((2, YPAD, D_H), jnp.float32),
    )(y0, W0, bt[:, :, None, :])


def _zadd_body(z_ref, y_ref, w_ref, zo_ref):
    zo_ref[0] = z_ref[0] + jnp.dot(y_ref[...], w_ref[0],
                                   preferred_element_type=jnp.float32)


def _zadd(z, y, W):
    return pl.pallas_call(
        _zadd_body,
        grid=(2, NBLK2),
        in_specs=[
            pl.BlockSpec((1, BLK2, D_H), lambda t, i: (t, i, 0)),
            pl.BlockSpec((BLK2, D_H), lambda t, i: (t * NBLK2 + i, 0)),
            pl.BlockSpec((1, D_H, D_H), lambda t, i: (t, 0, 0)),
        ],
        out_specs=pl.BlockSpec((1, BLK2, D_H), lambda t, i: (t, i, 0)),
        out_shape=jax.ShapeDtypeStruct((2, YPAD, D_H), jnp.float32),
    )(z, y, W)


def _final_body(h_ref, z_ref, y_ref, w_ref, ho_ref, sum_ref, sq_ref):
    i = pl.program_id(1)
    acc = h_ref[0] + z_ref[0] + jnp.dot(y_ref[...], w_ref[0],
                                        preferred_element_type=jnp.float32)
    ho_ref[0] = acc
    rows = i * BLK2 + lax.broadcasted_iota(jnp.int32, (BLK2, 1), 0)
    m = rows < N
    ps = jnp.sum(jnp.where(m, acc, 0.0), 0)[None, None, :]
    pq = jnp.sum(jnp.where(m, acc * acc, 0.0), 0)[None, None, :]

    @pl.when(i == 0)
    def _():
        sum_ref[...] = ps
        sq_ref[...] = pq

    @pl.when(i > 0)
    def _():
        sum_ref[...] += ps
        sq_ref[...] += pq


def _final(h, z, y4, W4):
    return pl.pallas_call(
        _final_body,
        grid=(2, NBLK2),
        in_specs=[
            pl.BlockSpec((1, BLK2, D_H), lambda t, i: (t, i, 0)),
            pl.BlockSpec((1, BLK2, D_H), lambda t, i: (t, i, 0)),
            pl.BlockSpec((BLK2, D_H), lambda t, i: (t * NBLK2 + i, 0)),
            pl.BlockSpec((1, D_H, D_H), lambda t, i: (t, 0, 0)),
        ],
        out_specs=[
            pl.BlockSpec((1, BLK2, D_H), lambda t, i: (t, i, 0)),
            pl.BlockSpec((1, 1, D_H), lambda t, i: (t, 0, 0)),
            pl.BlockSpec((1, 1, D_H), lambda t, i: (t, 0, 0)),
        ],
        out_shape=[
            jax.ShapeDtypeStruct((2, YPAD, D_H), jnp.float32),
            jax.ShapeDtypeStruct((2, 1, D_H), jnp.float32),
            jax.ShapeDtypeStruct((2, 1, D_H), jnp.float32),
        ],
    )(h, z, y4, W4)


def _comb_body(h_ref, y0, y1, y2, y3, y4, w_ref, b_ref, ho_ref, sum_ref, sq_ref):
    i = pl.program_id(1)
    acc = h_ref[0]
    for k, y in enumerate((y0, y1, y2, y3, y4)):
        acc = acc + jnp.dot(y[...], w_ref[k, 0], preferred_element_type=jnp.float32)
    acc = acc + jnp.sum(b_ref[...], axis=0)[0]
    ho_ref[0] = acc
    rows = i * BLK2 + lax.broadcasted_iota(jnp.int32, (BLK2, 1), 0)
    m = rows < N
    ps = jnp.sum(jnp.where(m, acc, 0.0), 0)[None, None, :]
    pq = jnp.sum(jnp.where(m, acc * acc, 0.0), 0)[None, None, :]

    @pl.when(i == 0)
    def _():
        sum_ref[...] = ps
        sq_ref[...] = pq

    @pl.when(i > 0)
    def _():
        sum_ref[...] += ps
        sq_ref[...] += pq


def _combine(h, ys, Wt, bt):
    yflat = pl.BlockSpec((BLK2, D_H), lambda t, i: (t * NBLK2 + i, 0))
    return pl.pallas_call(
        _comb_body,
        grid=(2, NBLK2),
        in_specs=[pl.BlockSpec((1, BLK2, D_H), lambda t, i: (t, i, 0))]
        + [yflat] * 5
        + [
            pl.BlockSpec((N_TAPS + 1, 1, D_H, D_H), lambda t, i: (0, t, 0, 0)),
            pl.BlockSpec((N_TAPS + 1, 1, 1, D_H), lambda t, i: (0, t, 0, 0)),
        ],
        out_specs=[
            pl.BlockSpec((1, BLK2, D_H), lambda t, i: (t, i, 0)),
            pl.BlockSpec((1, 1, D_H), lambda t, i: (t, 0, 0)),
            pl.BlockSpec((1, 1, D_H), lambda t, i: (t, 0, 0)),
        ],
        out_shape=[
            jax.ShapeDtypeStruct((2, YPAD, D_H), jnp.float32),
            jax.ShapeDtypeStruct((2, 1, D_H), jnp.float32),
            jax.ShapeDtypeStruct((2, 1, D_H), jnp.float32),
        ],
    )(h, *ys, Wt, bt[:, :, None, :])


def _readout_body(h_ref, w_ref, b_ref, o_ref):
    o_ref[0] = jnp.dot(h_ref[0], w_ref[0],
                       preferred_element_type=jnp.float32) + b_ref[0, 0]


def _readout(h, W_out, b_out):
    return pl.pallas_call(
        _readout_body,
        grid=(2, NBLK2),
        in_specs=[
            pl.BlockSpec((1, BLK2, D_H), lambda t, i: (t, i, 0)),
            pl.BlockSpec((1, D_H, D_IN), lambda t, i: (t, 0, 0)),
            pl.BlockSpec((1, 1, D_IN), lambda t, i: (t, 0, 0)),
        ],
        out_specs=pl.BlockSpec((1, BLK2, D_IN), lambda t, i: (t, i, 0)),
        out_shape=jax.ShapeDtypeStruct((2, N, D_IN), jnp.float32),
    )(h, W_out, b_out[:, None, :])


_sc_mesh = plsc.VectorSubcoreMesh(core_axis_name="c", subcore_axis_name="s")


@functools.partial(
    pl.kernel,
    out_type=jax.ShapeDtypeStruct((2 * YPAD, D_H), jnp.float32),
    mesh=_sc_mesh,
    scratch_types=[
        pltpu.VMEM((2, 2, NB, CH), jnp.int32),
        pltpu.VMEM((NB, CH, D_H), jnp.float32),
        pltpu.VMEM_SHARED((YPAD, D_H), jnp.float32),
    ] + [pltpu.SemaphoreType.DMA] * (2 * NB + 1),
    compiler_params=pltpu.CompilerParams(use_tc_tiling_on_sc=False),
)
def _seg_pair(y2_hbm, srcx_hbm, dstx_hbm, zeros_hbm, out_hbm,
              idx, rows, acc, *sems):
    gsem = sems[:NB]
    ssem = sems[NB:2 * NB]
    isem = sems[2 * NB]
    c = lax.axis_index("c")
    s = lax.axis_index("s")
    row0 = s * RPT
    crow0 = c * NROW2D + s * NCH_TILE

    def idx_copies(g):
        p = g & 1
        sl = pl.ds(crow0 + g * NB, NB)
        return (
            pltpu.make_async_copy(srcx_hbm.at[sl, :], idx.at[p, 0], isem),
            pltpu.make_async_copy(dstx_hbm.at[sl, :], idx.at[p, 1], isem),
        )

    def idx_start(g):
        for d in idx_copies(g):
            d.start()

    def idx_wait(g):
        for d in idx_copies(g):
            d.wait()

    def fire_gathers(g):
        p = g & 1
        for b in range(NB):
            pltpu.async_copy(y2_hbm.at[idx.at[p, 0, b]], rows.at[b], gsem[b])

    def wait_and_scatter(g):
        p = g & 1
        for b in range(NB):
            pltpu.make_async_copy(
                y2_hbm.at[idx.at[p, 0, b]], rows.at[b], gsem[b]).wait()
            pltpu.async_copy(rows.at[b], acc.at[idx.at[p, 1, b]], ssem[b],
                             add=True)

    def drain_scatters(g):
        p = g & 1
        for b in range(NB):
            pltpu.make_async_copy(
                rows.at[b], acc.at[idx.at[p, 1, b]], ssem[b]).wait()

    idx_start(0)
    pltpu.sync_copy(zeros_hbm.at[pl.ds(row0, RPT), :],
                    acc.at[pl.ds(row0, RPT), :])
    plsc.subcore_barrier()

    idx_wait(0)
    idx_start(1)
    fire_gathers(0)
    wait_and_scatter(0)

    @pl.loop(1, NGRP - 1)
    def _(g):
        drain_scatters(g - 1)
        idx_wait(g)
        idx_start(g + 1)
        fire_gathers(g)
        wait_and_scatter(g)

    drain_scatters(NGRP - 2)
    idx_wait(NGRP - 1)
    fire_gathers(NGRP - 1)
    wait_and_scatter(NGRP - 1)
    drain_scatters(NGRP - 1)

    plsc.subcore_barrier()

    pltpu.sync_copy(acc.at[pl.ds(row0, RPT), :],
                    out_hbm.at[pl.ds(c * YPAD + row0, RPT), :])


def _seg_tap(y2, srcx, dstx, zeros):
    return _seg_pair(y2, srcx, dstx, zeros)


def kernel(x_a, x_b, edge_index_ab, edge_index_ba, W_in, b_in, W_taps,
           b_taps, gamma, beta, W_out, b_out):
    eab = lax.optimization_barrier(edge_index_ab.reshape(2 * E))
    eba = lax.optimization_barrier(edge_index_ba.reshape(2 * E))
    srcx = jnp.concatenate([eba[:E] + YPAD,
                            eab[:E]]).reshape(2 * NROW2D, CH)
    dstx = jnp.concatenate([eba[E:], eab[E:]]).reshape(2 * NROW2D, CH)
    zeros = jnp.zeros((YPAD, D_H), jnp.float32)

    h, ssum, ssq = _readin(x_a, x_b, W_in, b_in)
    for l in range(N_LAYERS):
        y = _norm_leaky(h, ssum, ssq, gamma[l], beta[l])
        z = None
        for i in range(N_TAPS):
            yn = _seg_tap(y, srcx, dstx, zeros)
            z = _zinit(y, W_taps[l, 0], b_taps[l]) if z is None \
                else _zadd(z, y, W_taps[l, i])
            y = yn
        h, ssum, ssq = _final(h, z, y, W_taps[l, N_TAPS])
    return _readout(h, W_out, b_out)


# speedup vs baseline: 1.0508x; 1.0153x over previous
import functools

import jax
import jax.numpy as jnp
from jax import lax
from jax.experimental import pallas as pl
from jax.experimental.pallas import tpu as pltpu
from jax.experimental.pallas import tpu_sc as plsc

N = 50000
E = 1600000
D_IN = 128
D_H = 32
N_TAPS = 4
N_LAYERS = 2

BLK = 2000
NBLK = N // BLK
YPAD = 50048
BLK2 = 3128
NBLK2 = YPAD // BLK2

NTILES = 16
CH = 100
NB = 8
NROW2D = E // CH
NCH_TILE = NROW2D // NTILES
NGRP = NCH_TILE // NB
RPT = YPAD // NTILES


def _readin_body(xa_ref, xb_ref, w_ref, b_ref, h_ref, sum_ref, sq_ref):
    i = pl.program_id(0)
    ha = jnp.dot(xa_ref[...], w_ref[0], preferred_element_type=jnp.float32) + b_ref[0]
    hb = jnp.dot(xb_ref[...], w_ref[1], preferred_element_type=jnp.float32) + b_ref[1]
    h_ref[0] = ha
    h_ref[1] = hb
    ps = jnp.stack([jnp.sum(ha, 0), jnp.sum(hb, 0)])[:, None, :]
    pq = jnp.stack([jnp.sum(ha * ha, 0), jnp.sum(hb * hb, 0)])[:, None, :]

    @pl.when(i == 0)
    def _():
        sum_ref[...] = ps
        sq_ref[...] = pq

    @pl.when(i > 0)
    def _():
        sum_ref[...] += ps
        sq_ref[...] += pq


def _readin(x_a, x_b, W_in, b_in):
    return pl.pallas_call(
        _readin_body,
        grid=(NBLK,),
        in_specs=[
            pl.BlockSpec((BLK, D_IN), lambda i: (i, 0)),
            pl.BlockSpec((BLK, D_IN), lambda i: (i, 0)),
            pl.BlockSpec((2, D_IN, D_H), lambda i: (0, 0, 0)),
            pl.BlockSpec((2, D_H), lambda i: (0, 0)),
        ],
        out_specs=[
            pl.BlockSpec((2, BLK, D_H), lambda i: (0, i, 0)),
            pl.BlockSpec((2, 1, D_H), lambda i: (0, 0, 0)),
            pl.BlockSpec((2, 1, D_H), lambda i: (0, 0, 0)),
        ],
        out_shape=[
            jax.ShapeDtypeStruct((2, YPAD, D_H), jnp.float32),
            jax.ShapeDtypeStruct((2, 1, D_H), jnp.float32),
            jax.ShapeDtypeStruct((2, 1, D_H), jnp.float32),
        ],
    )(x_a, x_b, W_in, b_in)


def _norm_body(h_ref, sum_ref, sq_ref, g_ref, bt_ref, y_ref):
    mu = sum_ref[0] / N
    var = sq_ref[0] / N - mu * mu
    inv = lax.rsqrt(var + 1e-5)
    v = (h_ref[0] - mu) * inv * g_ref[0] + bt_ref[0]
    y_ref[...] = jnp.where(v >= 0, v, 0.01 * v)


def _norm_leaky(h, ssum, ssq, g, bt):
    return pl.pallas_call(
        _norm_body,
        grid=(2, NBLK2),
        in_specs=[
            pl.BlockSpec((1, BLK2, D_H), lambda t, i: (t, i, 0)),
            pl.BlockSpec((1, 1, D_H), lambda t, i: (t, 0, 0)),
            pl.BlockSpec((1, 1, D_H), lambda t, i: (t, 0, 0)),
            pl.BlockSpec((1, 1, D_H), lambda t, i: (t, 0, 0)),
            pl.BlockSpec((1, 1, D_H), lambda t, i: (t, 0, 0)),
        ],
        out_specs=pl.BlockSpec((BLK2, D_H), lambda t, i: (t * NBLK2 + i, 0)),
        out_shape=jax.ShapeDtypeStruct((2 * YPAD, D_H), jnp.float32),
    )(h, ssum, ssq, g[:, None, :], bt[:, None, :])


def _comb_body(h_ref, y0, y1, y2, y3, y4, w_ref, b_ref, ho_ref, sum_ref, sq_ref):
    i = pl.program_id(1)
    acc = h_ref[0]
    for k, y in enumerate((y0, y1, y2, y3, y4)):
        acc = acc + jnp.dot(y[...], w_ref[k, 0], preferred_element_type=jnp.float32)
    acc = acc + jnp.sum(b_ref[...], axis=0)[0]
    ho_ref[0] = acc
    rows = i * BLK2 + lax.broadcasted_iota(jnp.int32, (BLK2, 1), 0)
    m = rows < N
    ps = jnp.sum(jnp.where(m, acc, 0.0), 0)[None, None, :]
    pq = jnp.sum(jnp.where(m, acc * acc, 0.0), 0)[None, None, :]

    @pl.when(i == 0)
    def _():
        sum_ref[...] = ps
        sq_ref[...] = pq

    @pl.when(i > 0)
    def _():
        sum_ref[...] += ps
        sq_ref[...] += pq


def _combine(h, ys, Wt, bt):
    yflat = pl.BlockSpec((BLK2, D_H), lambda t, i: (t * NBLK2 + i, 0))
    return pl.pallas_call(
        _comb_body,
        grid=(2, NBLK2),
        in_specs=[pl.BlockSpec((1, BLK2, D_H), lambda t, i: (t, i, 0))]
        + [yflat] * 5
        + [
            pl.BlockSpec((N_TAPS + 1, 1, D_H, D_H), lambda t, i: (0, t, 0, 0)),
            pl.BlockSpec((N_TAPS + 1, 1, 1, D_H), lambda t, i: (0, t, 0, 0)),
        ],
        out_specs=[
            pl.BlockSpec((1, BLK2, D_H), lambda t, i: (t, i, 0)),
            pl.BlockSpec((1, 1, D_H), lambda t, i: (t, 0, 0)),
            pl.BlockSpec((1, 1, D_H), lambda t, i: (t, 0, 0)),
        ],
        out_shape=[
            jax.ShapeDtypeStruct((2, YPAD, D_H), jnp.float32),
            jax.ShapeDtypeStruct((2, 1, D_H), jnp.float32),
            jax.ShapeDtypeStruct((2, 1, D_H), jnp.float32),
        ],
    )(h, *ys, Wt, bt[:, :, None, :])


def _readout_body(h_ref, w_ref, b_ref, o_ref):
    o_ref[0] = jnp.dot(h_ref[0], w_ref[0],
                       preferred_element_type=jnp.float32) + b_ref[0, 0]


def _readout(h, W_out, b_out):
    return pl.pallas_call(
        _readout_body,
        grid=(2, NBLK2),
        in_specs=[
            pl.BlockSpec((1, BLK2, D_H), lambda t, i: (t, i, 0)),
            pl.BlockSpec((1, D_H, D_IN), lambda t, i: (t, 0, 0)),
            pl.BlockSpec((1, 1, D_IN), lambda t, i: (t, 0, 0)),
        ],
        out_specs=pl.BlockSpec((1, BLK2, D_IN), lambda t, i: (t, i, 0)),
        out_shape=jax.ShapeDtypeStruct((2, N, D_IN), jnp.float32),
    )(h, W_out, b_out[:, None, :])


_sc_mesh = plsc.VectorSubcoreMesh(core_axis_name="c", subcore_axis_name="s")


@functools.partial(
    pl.kernel,
    out_type=jax.ShapeDtypeStruct((2 * YPAD, D_H), jnp.float32),
    mesh=_sc_mesh,
    scratch_types=[
        pltpu.VMEM((2, 2, NB, CH), jnp.int32),
        pltpu.VMEM((NB, CH, D_H), jnp.float32),
        pltpu.VMEM_SHARED((YPAD, D_H), jnp.float32),
    ] + [pltpu.SemaphoreType.DMA] * (2 * NB + 1),
    compiler_params=pltpu.CompilerParams(use_tc_tiling_on_sc=False),
)
def _seg_pair(y2_hbm, srcx_hbm, dstx_hbm, zeros_hbm, out_hbm,
              idx, rows, acc, *sems):
    gsem = sems[:NB]
    ssem = sems[NB:2 * NB]
    isem = sems[2 * NB]
    c = lax.axis_index("c")
    s = lax.axis_index("s")
    row0 = s * RPT
    crow0 = c * NROW2D + s * NCH_TILE

    def idx_copies(g):
        p = g & 1
        sl = pl.ds(crow0 + g * NB, NB)
        return (
            pltpu.make_async_copy(srcx_hbm.at[sl, :], idx.at[p, 0], isem),
            pltpu.make_async_copy(dstx_hbm.at[sl, :], idx.at[p, 1], isem),
        )

    def idx_start(g):
        for d in idx_copies(g):
            d.start()

    def idx_wait(g):
        for d in idx_copies(g):
            d.wait()

    def fire_gathers(g):
        p = g & 1
        for b in range(NB):
            pltpu.async_copy(y2_hbm.at[idx.at[p, 0, b]], rows.at[b], gsem[b])

    def wait_and_scatter(g):
        p = g & 1
        for b in range(NB):
            pltpu.make_async_copy(
                y2_hbm.at[idx.at[p, 0, b]], rows.at[b], gsem[b]).wait()
            pltpu.async_copy(rows.at[b], acc.at[idx.at[p, 1, b]], ssem[b],
                             add=True)

    def drain_scatters(g):
        p = g & 1
        for b in range(NB):
            pltpu.make_async_copy(
                rows.at[b], acc.at[idx.at[p, 1, b]], ssem[b]).wait()

    idx_start(0)
    pltpu.sync_copy(zeros_hbm.at[pl.ds(row0, RPT), :],
                    acc.at[pl.ds(row0, RPT), :])
    plsc.subcore_barrier()

    idx_wait(0)
    idx_start(1)
    fire_gathers(0)
    wait_and_scatter(0)

    @pl.loop(1, NGRP - 1)
    def _(g):
        drain_scatters(g - 1)
        idx_wait(g)
        idx_start(g + 1)
        fire_gathers(g)
        wait_and_scatter(g)

    drain_scatters(NGRP - 2)
    idx_wait(NGRP - 1)
    fire_gathers(NGRP - 1)
    wait_and_scatter(NGRP - 1)
    drain_scatters(NGRP - 1)

    plsc.subcore_barrier()

    pltpu.sync_copy(acc.at[pl.ds(row0, RPT), :],
                    out_hbm.at[pl.ds(c * YPAD + row0, RPT), :])


def _seg_tap(y2, srcx, dstx, zeros):
    return _seg_pair(y2, srcx, dstx, zeros)


def kernel(x_a, x_b, edge_index_ab, edge_index_ba, W_in, b_in, W_taps,
           b_taps, gamma, beta, W_out, b_out):
    eab = lax.optimization_barrier(edge_index_ab.reshape(2 * E))
    eba = lax.optimization_barrier(edge_index_ba.reshape(2 * E))
    srcx = jnp.concatenate([eba[:E] + YPAD,
                            eab[:E]]).reshape(2 * NROW2D, CH)
    dstx = jnp.concatenate([eba[E:], eab[E:]]).reshape(2 * NROW2D, CH)
    zeros = jnp.zeros((YPAD, D_H), jnp.float32)

    h, ssum, ssq = _readin(x_a, x_b, W_in, b_in)
    for l in range(N_LAYERS):
        y = _norm_leaky(h, ssum, ssq, gamma[l], beta[l])
        ys = [y]
        for _ in range(N_TAPS):
            ys.append(_seg_tap(ys[-1], srcx, dstx, zeros))
        h, ssum, ssq = _combine(h, ys, W_taps[l], b_taps[l])
    return _readout(h, W_out, b_out)
